# baseline (speedup 1.0000x reference)
import functools

import jax
import jax.numpy as jnp
from jax import lax
from jax.experimental import pallas as pl
from jax.experimental.pallas import tpu as pltpu
from jax.experimental.pallas import tpu_sc as plsc

_S, _M, _E, _F = 2048, 2048, 8, 8192
_CAP = _S // _E
_FB = 512
_BIG = 1 << 30

_NC, _NS = 2, 16
_NW = _NC * _NS
_RW = _S // _NW
_CH = 16


def _shift_right_lanes(a, d):
    pad = jnp.zeros((a.shape[0], d), a.dtype)
    return jnp.concatenate([pad, a[:, : a.shape[1] - d]], axis=1)


def _gating_body(x_ref, wg_ref, src_ref, gslot_ref, dstc_ref, laux_ref):
    x = x_ref[...]
    wg = wg_ref[...]
    lt = lax.dot_general(wg, x, (((0,), (1,)), ((), ())),
                         preferred_element_type=jnp.float32)
    mx = jnp.max(lt, axis=0, keepdims=True)
    sub = lax.broadcasted_iota(jnp.int32, (_E, _S), 0)
    expert = jnp.min(jnp.where(lt == mx, sub, _E), axis=0, keepdims=True)
    mask1 = (sub == expert).astype(jnp.float32)
    ex = jnp.exp(lt - mx)
    gates = ex / jnp.sum(ex, axis=0, keepdims=True)

    me = jnp.mean(gates, axis=1, keepdims=True)
    ce = jnp.mean(mask1, axis=1, keepdims=True)
    laux_ref[...] = jnp.sum(me * ce).reshape(1, 1) * float(_E)

    c = mask1
    d = 1
    while d < _S:
        c = c + _shift_right_lanes(c, d)
        d *= 2
    loc = c - 1.0
    maskc = mask1 * (loc < float(_CAP)).astype(jnp.float32)
    locs = jnp.sum(loc * maskc, axis=0, keepdims=True)
    gate_s = jnp.sum(gates * maskc, axis=0, keepdims=True)
    kept = jnp.sum(maskc, axis=0, keepdims=True)
    slot = expert * _CAP + locs.astype(jnp.int32)

    tok = lax.broadcasted_iota(jnp.int32, (1, _S), 1).astype(jnp.float32)
    slot_m = jnp.where(kept > 0.0, slot, -1)
    zcand = jnp.int32(_BIG)
    for b in range(_E):
        sid = b * _CAP + lax.broadcasted_iota(jnp.int32, (_CAP, 1), 0)
        eq = (sid == slot_m).astype(jnp.float32)
        src_ref[b * _CAP:(b + 1) * _CAP, :] = jnp.sum(
            eq * tok, axis=1, keepdims=True).astype(jnp.int32)
        gslot_ref[b * _CAP:(b + 1) * _CAP, :] = jnp.sum(
            eq * gate_s, axis=1, keepdims=True)
        fill = jnp.sum(eq, axis=1, keepdims=True)
        zcand = jnp.minimum(zcand, jnp.min(jnp.where(fill < 0.5, sid, _BIG)))
    zslot = jnp.where(zcand >= _BIG, 0, zcand)
    dstc_ref[...] = jnp.where(kept > 0.0, slot, zslot)


def _ffn_body(gs_ref, disp_ref, w1_ref, w2_ref, out_ref, acc_ref):
    f = pl.program_id(1)
    nf = pl.num_programs(1)
    xb = disp_ref[...].astype(jnp.bfloat16)
    w1b = w1_ref[0].astype(jnp.bfloat16)
    h = jnp.dot(xb, w1b, preferred_element_type=jnp.float32)
    h = jax.nn.gelu(h)
    w2b = w2_ref[0].astype(jnp.bfloat16)
    part = jnp.dot(h.astype(jnp.bfloat16), w2b,
                   preferred_element_type=jnp.float32)

    @pl.when(f == 0)
    def _():
        acc_ref[...] = part

    @pl.when(f > 0)
    def _():
        acc_ref[...] += part

    @pl.when(f == nf - 1)
    def _():
        out_ref[...] = acc_ref[...] * gs_ref[...]


def _sc_gather_rows(table, idx):
    mesh = plsc.VectorSubcoreMesh(core_axis_name="c", subcore_axis_name="s")
    nch = _RW // _CH

    @functools.partial(
        pl.kernel,
        out_type=jax.ShapeDtypeStruct((_S, _M), jnp.float32),
        mesh=mesh,
        scratch_types=[
            pltpu.VMEM((_RW,), jnp.int32),
            pltpu.VMEM((_CH, _M), jnp.float32),
            pltpu.VMEM((_CH, _M), jnp.float32),
            pltpu.SemaphoreType.DMA,
            pltpu.SemaphoreType.DMA,
            pltpu.SemaphoreType.DMA,
            pltpu.SemaphoreType.DMA,
        ],
    )
    def k(table_hbm, idx_hbm, out_hbm, ix, b0, b1, sg0, sg1, sw0, sw1):
        wid = lax.axis_index("s") * _NC + lax.axis_index("c")
        base = wid * _RW
        bufs, sgs, sws = (b0, b1), (sg0, sg1), (sw0, sw1)
        pltpu.sync_copy(idx_hbm.at[pl.ds(base, _RW)], ix)

        def gather(c):
            return pltpu.async_copy(
                table_hbm.at[ix.at[pl.ds(c * _CH, _CH)]], bufs[c % 2], sgs[c % 2])

        gh = {c: gather(c) for c in range(min(2, nch))}
        wh = {}
        for c in range(nch):
            p = c % 2
            gh[c].wait()
            wh[c] = pltpu.async_copy(
                bufs[p], out_hbm.at[pl.ds(base + c * _CH, _CH)], sws[p])
            if c + 2 < nch:
                wh[c].wait()
                gh[c + 2] = gather(c + 2)
        for c in (nch - 2, nch - 1):
            wh[c].wait()

    return k(table, idx)


def _gating_call(x, wg):
    return pl.pallas_call(
        _gating_body,
        out_shape=[
            jax.ShapeDtypeStruct((_S, 1), jnp.int32),
            jax.ShapeDtypeStruct((_S, 1), jnp.float32),
            jax.ShapeDtypeStruct((1, _S), jnp.int32),
            jax.ShapeDtypeStruct((1, 1), jnp.float32),
        ],
    )(x, wg)


def _ffn_call(gslot, disp, w1, w2):
    nf = _F // _FB
    return pl.pallas_call(
        _ffn_body,
        grid=(_E, nf),
        in_specs=[
            pl.BlockSpec((_CAP, 1), lambda e, f: (e, 0)),
            pl.BlockSpec((_CAP, _M), lambda e, f: (e, 0)),
            pl.BlockSpec((1, _M, _FB), lambda e, f: (e, 0, f)),
            pl.BlockSpec((1, _FB, _M), lambda e, f: (e, f, 0)),
        ],
        out_specs=pl.BlockSpec((_CAP, _M), lambda e, f: (e, 0)),
        out_shape=jax.ShapeDtypeStruct((_S, _M), jnp.float32),
        scratch_shapes=[pltpu.VMEM((_CAP, _M), jnp.float32)],
        compiler_params=pltpu.CompilerParams(
            vmem_limit_bytes=63 * 1024 * 1024),
    )(gslot, disp, w1, w2)


def kernel(hidden_states, wg, w1, w2):
    x = hidden_states.reshape(-1, _M)
    src, gslot, dstc, laux = _gating_call(x, wg)
    disp = _sc_gather_rows(x, src.reshape(_S))
    eout = _ffn_call(gslot, disp, w1, w2)
    out = _sc_gather_rows(eout, dstc.reshape(_S))
    return out, laux.reshape(())


# speedup vs baseline: 1.0550x; 1.0550x over previous
import functools

import jax
import jax.numpy as jnp
from jax import lax
from jax.experimental import pallas as pl
from jax.experimental.pallas import tpu as pltpu
from jax.experimental.pallas import tpu_sc as plsc

_S, _M, _E, _F = 2048, 2048, 8, 8192
_CAP = _S // _E
_FB = 1024
_BIG = 1 << 30

_NC, _NS = 2, 16
_NW = _NC * _NS
_RW = _S // _NW
_CH = 16


def _shift_right_lanes(a, d):
    pad = jnp.zeros((a.shape[0], d), a.dtype)
    return jnp.concatenate([pad, a[:, : a.shape[1] - d]], axis=1)


def _cumsum_lanes(a):
    c = a
    d = 1
    while d < a.shape[1]:
        c = c + _shift_right_lanes(c, d)
        d *= 2
    return c


def _cumsum_sublanes(a):
    c = a
    d = 1
    while d < a.shape[0]:
        pad = jnp.zeros((d, a.shape[1]), a.dtype)
        c = c + jnp.concatenate([pad, c[: a.shape[0] - d, :]], axis=0)
        d *= 2
    return c


def _gating_body(x_ref, wg_ref, src_ref, gslot_ref, scat_ref, laux_ref):
    x = x_ref[...]
    wg = wg_ref[...]
    lt = lax.dot_general(wg, x, (((0,), (1,)), ((), ())),
                         preferred_element_type=jnp.float32)
    mx = jnp.max(lt, axis=0, keepdims=True)
    sub = lax.broadcasted_iota(jnp.int32, (_E, _S), 0)
    expert = jnp.min(jnp.where(lt == mx, sub, _E), axis=0, keepdims=True)
    mask1 = (sub == expert).astype(jnp.float32)
    ex = jnp.exp(lt - mx)
    gates = ex / jnp.sum(ex, axis=0, keepdims=True)

    me = jnp.mean(gates, axis=1, keepdims=True)
    ce = jnp.mean(mask1, axis=1, keepdims=True)
    laux_ref[...] = jnp.sum(me * ce).reshape(1, 1) * float(_E)

    loc = _cumsum_lanes(mask1) - 1.0
    maskc = mask1 * (loc < float(_CAP)).astype(jnp.float32)
    locs = jnp.sum(loc * maskc, axis=0, keepdims=True)
    gate_s = jnp.sum(gates * maskc, axis=0, keepdims=True)
    kept = jnp.sum(maskc, axis=0, keepdims=True)
    slot = expert * _CAP + locs.astype(jnp.int32)

    drop = 1.0 - kept
    drank = _cumsum_lanes(drop) - 1.0

    tok = lax.broadcasted_iota(jnp.int32, (1, _S), 1).astype(jnp.float32)
    slot_m = jnp.where(kept > 0.0, slot, -1)
    carry = 0.0
    for b in range(_E):
        sid = b * _CAP + lax.broadcasted_iota(jnp.int32, (_CAP, 1), 0)
        eq = (sid == slot_m).astype(jnp.float32)
        srcb = jnp.sum(eq * tok, axis=1, keepdims=True)
        gslot_ref[b * _CAP:(b + 1) * _CAP, :] = jnp.sum(
            eq * gate_s, axis=1, keepdims=True)
        fill = jnp.sum(eq, axis=1, keepdims=True)
        unf = 1.0 - fill
        q = carry + _cumsum_sublanes(unf) - 1.0
        carry = carry + jnp.sum(unf)
        eq2 = ((q == drank) * unf * drop).astype(jnp.float32)
        dstb = jnp.sum(eq2 * tok, axis=1, keepdims=True)
        scatb = jnp.where(fill > 0.0, srcb, dstb)
        src_ref[b * _CAP:(b + 1) * _CAP, :] = srcb.astype(jnp.int32)
        scat_ref[b * _CAP:(b + 1) * _CAP, :] = scatb.astype(jnp.int32)


def _ffn_body(scat_sp, gs_ref, disp_ref, w1_ref, w2_ref, out_ref,
              acc0, acc1, sem0, sem1):
    e = pl.program_id(0)
    f = pl.program_id(1)
    nf = pl.num_programs(1)
    even = lax.rem(e, 2) == 0

    @pl.when((f == 0) & (e >= 2) & even)
    def _():
        pltpu.make_async_copy(acc0, out_ref.at[pl.ds(0, _CAP)], sem0).wait()

    @pl.when((f == 0) & (e >= 2) & jnp.logical_not(even))
    def _():
        pltpu.make_async_copy(acc1, out_ref.at[pl.ds(0, _CAP)], sem1).wait()

    xb = disp_ref[...].astype(jnp.bfloat16)
    w1b = w1_ref[0].astype(jnp.bfloat16)
    h = jnp.dot(xb, w1b, preferred_element_type=jnp.float32)
    h = jax.nn.gelu(h)
    w2b = w2_ref[0].astype(jnp.bfloat16)
    part = jnp.dot(h.astype(jnp.bfloat16), w2b,
                   preferred_element_type=jnp.float32)

    def acc_step(acc):
        @pl.when(f == 0)
        def _():
            acc[...] = part

        @pl.when((f > 0) & (f < nf - 1))
        def _():
            acc[...] += part

        @pl.when(f == nf - 1)
        def _():
            acc[...] = (acc[...] + part) * gs_ref[...]

    @pl.when(even)
    def _():
        acc_step(acc0)

    @pl.when(jnp.logical_not(even))
    def _():
        acc_step(acc1)

    def issue(acc, sem):
        def body(j, _):
            row = scat_sp[e * _CAP + j]
            pltpu.make_async_copy(
                acc.at[pl.ds(j, 1)], out_ref.at[pl.ds(row, 1)], sem).start()
            return 0

        lax.fori_loop(0, _CAP, body, 0)

    @pl.when((f == nf - 1) & even)
    def _():
        issue(acc0, sem0)

    @pl.when((f == nf - 1) & jnp.logical_not(even))
    def _():
        issue(acc1, sem1)

    @pl.when((f == nf - 1) & (e == pl.num_programs(0) - 1))
    def _():
        pltpu.make_async_copy(acc0, out_ref.at[pl.ds(0, _CAP)], sem0).wait()
        pltpu.make_async_copy(acc1, out_ref.at[pl.ds(0, _CAP)], sem1).wait()


def _sc_gather_rows(table, idx):
    mesh = plsc.VectorSubcoreMesh(core_axis_name="c", subcore_axis_name="s")
    nch = _RW // _CH

    @functools.partial(
        pl.kernel,
        out_type=jax.ShapeDtypeStruct((_S, _M), jnp.float32),
        mesh=mesh,
        scratch_types=[
            pltpu.VMEM((_RW,), jnp.int32),
            pltpu.VMEM((_CH, _M), jnp.float32),
            pltpu.VMEM((_CH, _M), jnp.float32),
            pltpu.SemaphoreType.DMA,
            pltpu.SemaphoreType.DMA,
            pltpu.SemaphoreType.DMA,
            pltpu.SemaphoreType.DMA,
        ],
    )
    def k(table_hbm, idx_hbm, out_hbm, ix, b0, b1, sg0, sg1, sw0, sw1):
        wid = lax.axis_index("s") * _NC + lax.axis_index("c")
        base = wid * _RW
        bufs, sgs, sws = (b0, b1), (sg0, sg1), (sw0, sw1)
        pltpu.sync_copy(idx_hbm.at[pl.ds(base, _RW)], ix)

        def gather(c):
            return pltpu.async_copy(
                table_hbm.at[ix.at[pl.ds(c * _CH, _CH)]], bufs[c % 2], sgs[c % 2])

        gh = {c: gather(c) for c in range(min(2, nch))}
        wh = {}
        for c in range(nch):
            p = c % 2
            gh[c].wait()
            wh[c] = pltpu.async_copy(
                bufs[p], out_hbm.at[pl.ds(base + c * _CH, _CH)], sws[p])
            if c + 2 < nch:
                wh[c].wait()
                gh[c + 2] = gather(c + 2)
        for c in (nch - 2, nch - 1):
            wh[c].wait()

    return k(table, idx)


def _gating_call(x, wg):
    return pl.pallas_call(
        _gating_body,
        out_shape=[
            jax.ShapeDtypeStruct((_S, 1), jnp.int32),
            jax.ShapeDtypeStruct((_S, 1), jnp.float32),
            jax.ShapeDtypeStruct((_S, 1), jnp.int32),
            jax.ShapeDtypeStruct((1, 1), jnp.float32),
        ],
    )(x, wg)


def _ffn_call(scat, gslot, disp, w1, w2):
    nf = _F // _FB
    grid_spec = pltpu.PrefetchScalarGridSpec(
        num_scalar_prefetch=1,
        grid=(_E, nf),
        in_specs=[
            pl.BlockSpec((_CAP, 1), lambda e, f, *_: (e, 0)),
            pl.BlockSpec((_CAP, _M), lambda e, f, *_: (e, 0)),
            pl.BlockSpec((1, _M, _FB), lambda e, f, *_: (e, 0, f)),
            pl.BlockSpec((1, _FB, _M), lambda e, f, *_: (e, f, 0)),
        ],
        out_specs=pl.BlockSpec(memory_space=pl.ANY),
        scratch_shapes=[
            pltpu.VMEM((_CAP, _M), jnp.float32),
            pltpu.VMEM((_CAP, _M), jnp.float32),
            pltpu.SemaphoreType.DMA,
            pltpu.SemaphoreType.DMA,
        ],
    )
    return pl.pallas_call(
        _ffn_body,
        grid_spec=grid_spec,
        out_shape=jax.ShapeDtypeStruct((_S, _M), jnp.float32),
        compiler_params=pltpu.CompilerParams(
            vmem_limit_bytes=63 * 1024 * 1024),
    )(scat, gslot, disp, w1, w2)


def kernel(hidden_states, wg, w1, w2):
    x = hidden_states.reshape(-1, _M)
    src, gslot, scat, laux = _gating_call(x, wg)
    disp = _sc_gather_rows(x, src.reshape(_S))
    out = _ffn_call(scat.reshape(_S), gslot, disp, w1, w2)
    return out, laux.reshape(())


# speedup vs baseline: 1.0868x; 1.0301x over previous
import functools

import jax
import jax.numpy as jnp
from jax import lax
from jax.experimental import pallas as pl
from jax.experimental.pallas import tpu as pltpu
from jax.experimental.pallas import tpu_sc as plsc

_S, _M, _E, _F = 2048, 2048, 8, 8192
_CAP = _S // _E
_FB = 1024
_BIG = 1 << 30

_NC, _NS = 2, 16
_NW = _NC * _NS
_RW = _S // _NW
_CH = 16


def _shift_right_lanes(a, d):
    pad = jnp.zeros((a.shape[0], d), a.dtype)
    return jnp.concatenate([pad, a[:, : a.shape[1] - d]], axis=1)


def _cumsum_lanes(a):
    c = a
    d = 1
    while d < a.shape[1]:
        c = c + _shift_right_lanes(c, d)
        d *= 2
    return c


def _cumsum_sublanes(a):
    c = a
    d = 1
    while d < a.shape[0]:
        pad = jnp.zeros((d, a.shape[1]), a.dtype)
        c = c + jnp.concatenate([pad, c[: a.shape[0] - d, :]], axis=0)
        d *= 2
    return c


def _gating_body(x_ref, wg_ref, src_ref, gslot_ref, scat_ref, laux_ref):
    x = x_ref[...]
    wg = wg_ref[...]
    lt = lax.dot_general(wg, x, (((0,), (1,)), ((), ())),
                         preferred_element_type=jnp.float32)
    mx = jnp.max(lt, axis=0, keepdims=True)
    sub = lax.broadcasted_iota(jnp.int32, (_E, _S), 0)
    expert = jnp.min(jnp.where(lt == mx, sub, _E), axis=0, keepdims=True)
    mask1 = (sub == expert).astype(jnp.float32)
    ex = jnp.exp(lt - mx)
    gates = ex / jnp.sum(ex, axis=0, keepdims=True)

    me = jnp.mean(gates, axis=1, keepdims=True)
    ce = jnp.mean(mask1, axis=1, keepdims=True)
    laux_ref[...] = jnp.sum(me * ce).reshape(1, 1) * float(_E)

    loc = _cumsum_lanes(mask1) - 1.0
    maskc = mask1 * (loc < float(_CAP)).astype(jnp.float32)
    locs = jnp.sum(loc * maskc, axis=0, keepdims=True)
    gate_s = jnp.sum(gates * maskc, axis=0, keepdims=True)
    kept = jnp.sum(maskc, axis=0, keepdims=True)
    slot = expert * _CAP + locs.astype(jnp.int32)

    drop = 1.0 - kept
    drank = _cumsum_lanes(drop) - 1.0

    tok = lax.broadcasted_iota(jnp.int32, (1, _S), 1).astype(jnp.float32)
    slot_m = jnp.where(kept > 0.0, slot, -1)
    carry = 0.0
    for b in range(_E):
        sid = b * _CAP + lax.broadcasted_iota(jnp.int32, (_CAP, 1), 0)
        eq = (sid == slot_m).astype(jnp.float32)
        srcb = jnp.sum(eq * tok, axis=1, keepdims=True)
        gslot_ref[b * _CAP:(b + 1) * _CAP, :] = jnp.sum(
            eq * gate_s, axis=1, keepdims=True)
        fill = jnp.sum(eq, axis=1, keepdims=True)
        unf = 1.0 - fill
        q = carry + _cumsum_sublanes(unf) - 1.0
        carry = carry + jnp.sum(unf)
        eq2 = ((q == drank) * unf * drop).astype(jnp.float32)
        dstb = jnp.sum(eq2 * tok, axis=1, keepdims=True)
        scatb = jnp.where(fill > 0.0, srcb, dstb)
        src_ref[b * _CAP:(b + 1) * _CAP, :] = srcb.astype(jnp.int32)
        scat_ref[b * _CAP:(b + 1) * _CAP, :] = scatb.astype(jnp.int32)


def _ffn_body(scat_sp, gs_ref, disp_ref, w1_ref, w2_ref, out_ref,
              acc0, acc1, sem0, sem1):
    e = pl.program_id(0)
    f = pl.program_id(1)
    nf = pl.num_programs(1)
    even = lax.rem(e, 2) == 0

    @pl.when((f == 0) & (e >= 2) & even)
    def _():
        pltpu.make_async_copy(acc0, out_ref.at[pl.ds(0, _CAP)], sem0).wait()

    @pl.when((f == 0) & (e >= 2) & jnp.logical_not(even))
    def _():
        pltpu.make_async_copy(acc1, out_ref.at[pl.ds(0, _CAP)], sem1).wait()

    xb = disp_ref[...].astype(jnp.bfloat16)
    w1b = w1_ref[0].astype(jnp.bfloat16)
    h = jnp.dot(xb, w1b, preferred_element_type=jnp.float32)
    h = jax.nn.gelu(h)
    w2b = w2_ref[0].astype(jnp.bfloat16)
    part = jnp.dot(h.astype(jnp.bfloat16), w2b,
                   preferred_element_type=jnp.float32)

    def acc_step(acc):
        @pl.when(f == 0)
        def _():
            acc[...] = part

        @pl.when((f > 0) & (f < nf - 1))
        def _():
            acc[...] += part

        @pl.when(f == nf - 1)
        def _():
            acc[...] = (acc[...] + part) * gs_ref[...]

    @pl.when(even)
    def _():
        acc_step(acc0)

    @pl.when(jnp.logical_not(even))
    def _():
        acc_step(acc1)

    def issue(acc, sem):
        base = e * _CAP
        for j in range(_CAP):
            row = scat_sp[base + j]
            pltpu.make_async_copy(
                acc.at[pl.ds(j, 1)], out_ref.at[pl.ds(row, 1)], sem).start()

    @pl.when((f == nf - 1) & even)
    def _():
        issue(acc0, sem0)

    @pl.when((f == nf - 1) & jnp.logical_not(even))
    def _():
        issue(acc1, sem1)

    @pl.when((f == nf - 1) & (e == pl.num_programs(0) - 1))
    def _():
        pltpu.make_async_copy(acc0, out_ref.at[pl.ds(0, _CAP)], sem0).wait()
        pltpu.make_async_copy(acc1, out_ref.at[pl.ds(0, _CAP)], sem1).wait()


def _sc_gather_rows(table, idx):
    mesh = plsc.VectorSubcoreMesh(core_axis_name="c", subcore_axis_name="s")
    nch = _RW // _CH

    @functools.partial(
        pl.kernel,
        out_type=jax.ShapeDtypeStruct((_S, _M), jnp.float32),
        mesh=mesh,
        scratch_types=[
            pltpu.VMEM((_RW,), jnp.int32),
            pltpu.VMEM((_CH, _M), jnp.float32),
            pltpu.VMEM((_CH, _M), jnp.float32),
            pltpu.SemaphoreType.DMA,
            pltpu.SemaphoreType.DMA,
            pltpu.SemaphoreType.DMA,
            pltpu.SemaphoreType.DMA,
        ],
    )
    def k(table_hbm, idx_hbm, out_hbm, ix, b0, b1, sg0, sg1, sw0, sw1):
        wid = lax.axis_index("s") * _NC + lax.axis_index("c")
        base = wid * _RW
        bufs, sgs, sws = (b0, b1), (sg0, sg1), (sw0, sw1)
        pltpu.sync_copy(idx_hbm.at[pl.ds(base, _RW)], ix)

        def gather(c):
            return pltpu.async_copy(
                table_hbm.at[ix.at[pl.ds(c * _CH, _CH)]], bufs[c % 2], sgs[c % 2])

        gh = {c: gather(c) for c in range(min(2, nch))}
        wh = {}
        for c in range(nch):
            p = c % 2
            gh[c].wait()
            wh[c] = pltpu.async_copy(
                bufs[p], out_hbm.at[pl.ds(base + c * _CH, _CH)], sws[p])
            if c + 2 < nch:
                wh[c].wait()
                gh[c + 2] = gather(c + 2)
        for c in (nch - 2, nch - 1):
            wh[c].wait()

    return k(table, idx)


def _gating_call(x, wg):
    return pl.pallas_call(
        _gating_body,
        out_shape=[
            jax.ShapeDtypeStruct((_S, 1), jnp.int32),
            jax.ShapeDtypeStruct((_S, 1), jnp.float32),
            jax.ShapeDtypeStruct((_S, 1), jnp.int32),
            jax.ShapeDtypeStruct((1, 1), jnp.float32),
        ],
    )(x, wg)


def _ffn_call(scat, gslot, disp, w1, w2):
    nf = _F // _FB
    grid_spec = pltpu.PrefetchScalarGridSpec(
        num_scalar_prefetch=1,
        grid=(_E, nf),
        in_specs=[
            pl.BlockSpec((_CAP, 1), lambda e, f, *_: (e, 0)),
            pl.BlockSpec((_CAP, _M), lambda e, f, *_: (e, 0)),
            pl.BlockSpec((1, _M, _FB), lambda e, f, *_: (e, 0, f)),
            pl.BlockSpec((1, _FB, _M), lambda e, f, *_: (e, f, 0)),
        ],
        out_specs=pl.BlockSpec(memory_space=pl.ANY),
        scratch_shapes=[
            pltpu.VMEM((_CAP, _M), jnp.float32),
            pltpu.VMEM((_CAP, _M), jnp.float32),
            pltpu.SemaphoreType.DMA,
            pltpu.SemaphoreType.DMA,
        ],
    )
    return pl.pallas_call(
        _ffn_body,
        grid_spec=grid_spec,
        out_shape=jax.ShapeDtypeStruct((_S, _M), jnp.float32),
        compiler_params=pltpu.CompilerParams(
            vmem_limit_bytes=63 * 1024 * 1024),
    )(scat, gslot, disp, w1, w2)


def kernel(hidden_states, wg, w1, w2):
    x = hidden_states.reshape(-1, _M)
    src, gslot, scat, laux = _gating_call(x, wg)
    disp = _sc_gather_rows(x, src.reshape(_S))
    out = _ffn_call(scat.reshape(_S), gslot, disp, w1, w2)
    return out, laux.reshape(())


# speedup vs baseline: 1.0925x; 1.0052x over previous
import functools

import jax
import jax.numpy as jnp
from jax import lax
from jax.experimental import pallas as pl
from jax.experimental.pallas import tpu as pltpu
from jax.experimental.pallas import tpu_sc as plsc

_S, _M, _E, _F = 2048, 2048, 8, 8192
_CAP = _S // _E
_FB = 1024
_BIG = 1 << 30

_NC, _NS = 2, 16
_NW = _NC * _NS
_RW = _S // _NW
_CH = 16


def _shift_right_lanes(a, d):
    pad = jnp.zeros((a.shape[0], d), a.dtype)
    return jnp.concatenate([pad, a[:, : a.shape[1] - d]], axis=1)


def _cumsum_lanes(a):
    c = a
    d = 1
    while d < a.shape[1]:
        c = c + _shift_right_lanes(c, d)
        d *= 2
    return c


def _cumsum_sublanes(a):
    c = a
    d = 1
    while d < a.shape[0]:
        pad = jnp.zeros((d, a.shape[1]), a.dtype)
        c = c + jnp.concatenate([pad, c[: a.shape[0] - d, :]], axis=0)
        d *= 2
    return c


def _gating_body(x_ref, wg_ref, src_ref, gslot_ref, scat_ref, laux_ref):
    x = x_ref[...]
    wg = wg_ref[...]
    lt = lax.dot_general(wg, x, (((0,), (1,)), ((), ())),
                         preferred_element_type=jnp.float32)
    mx = jnp.max(lt, axis=0, keepdims=True)
    sub = lax.broadcasted_iota(jnp.int32, (_E, _S), 0)
    expert = jnp.min(jnp.where(lt == mx, sub, _E), axis=0, keepdims=True)
    mask1 = (sub == expert).astype(jnp.float32)
    ex = jnp.exp(lt - mx)
    gates = ex / jnp.sum(ex, axis=0, keepdims=True)

    me = jnp.mean(gates, axis=1, keepdims=True)
    ce = jnp.mean(mask1, axis=1, keepdims=True)
    laux_ref[...] = jnp.sum(me * ce).reshape(1, 1) * float(_E)

    loc = _cumsum_lanes(mask1) - 1.0
    maskc = mask1 * (loc < float(_CAP)).astype(jnp.float32)
    locs = jnp.sum(loc * maskc, axis=0, keepdims=True)
    gate_s = jnp.sum(gates * maskc, axis=0, keepdims=True)
    kept = jnp.sum(maskc, axis=0, keepdims=True)
    slot = expert * _CAP + locs.astype(jnp.int32)

    drop = 1.0 - kept
    drank = jnp.where(drop > 0.0, _cumsum_lanes(drop) - 1.0, -3.0)

    tok = lax.broadcasted_iota(jnp.int32, (1, _S), 1).astype(jnp.float32)
    slot_m = jnp.where(kept > 0.0, slot, -1)
    carry = 0.0
    for b in range(_E):
        sid = b * _CAP + lax.broadcasted_iota(jnp.int32, (_CAP, 1), 0)
        eq = (sid == slot_m).astype(jnp.float32)
        srcb = jnp.sum(eq * tok, axis=1, keepdims=True)
        gslot_ref[b * _CAP:(b + 1) * _CAP, :] = jnp.sum(
            eq * gate_s, axis=1, keepdims=True)
        fill = jnp.sum(eq, axis=1, keepdims=True)
        unf = 1.0 - fill
        q = carry + _cumsum_sublanes(unf) - 1.0
        carry = carry + jnp.sum(unf)
        qm = jnp.where(unf > 0.0, q, -2.0)
        eq2 = (qm == drank).astype(jnp.float32)
        dstb = jnp.sum(eq2 * tok, axis=1, keepdims=True)
        scatb = jnp.where(fill > 0.0, srcb, dstb)
        src_ref[b * _CAP:(b + 1) * _CAP, :] = srcb.astype(jnp.int32)
        scat_ref[b * _CAP:(b + 1) * _CAP, :] = scatb.astype(jnp.int32)


def _ffn_body(scat_sp, gs_ref, disp_ref, w1_ref, w2_ref, out_ref,
              acc0, acc1, sem0, sem1):
    e = pl.program_id(0)
    f = pl.program_id(1)
    nf = pl.num_programs(1)
    even = lax.rem(e, 2) == 0

    @pl.when((f == 0) & (e >= 2) & even)
    def _():
        pltpu.make_async_copy(acc0, out_ref.at[pl.ds(0, _CAP)], sem0).wait()

    @pl.when((f == 0) & (e >= 2) & jnp.logical_not(even))
    def _():
        pltpu.make_async_copy(acc1, out_ref.at[pl.ds(0, _CAP)], sem1).wait()

    xb = disp_ref[...].astype(jnp.bfloat16)
    w1b = w1_ref[0].astype(jnp.bfloat16)
    h = jnp.dot(xb, w1b, preferred_element_type=jnp.float32)
    h = jax.nn.gelu(h)
    w2b = w2_ref[0].astype(jnp.bfloat16)
    part = jnp.dot(h.astype(jnp.bfloat16), w2b,
                   preferred_element_type=jnp.float32)

    def acc_step(acc):
        @pl.when(f == 0)
        def _():
            acc[...] = part

        @pl.when((f > 0) & (f < nf - 1))
        def _():
            acc[...] += part

        @pl.when(f == nf - 1)
        def _():
            acc[...] = (acc[...] + part) * gs_ref[...]

    @pl.when(even)
    def _():
        acc_step(acc0)

    @pl.when(jnp.logical_not(even))
    def _():
        acc_step(acc1)

    def issue(acc, sem):
        base = e * _CAP
        for j in range(_CAP):
            row = scat_sp[base + j]
            pltpu.make_async_copy(
                acc.at[pl.ds(j, 1)], out_ref.at[pl.ds(row, 1)], sem).start()

    @pl.when((f == nf - 1) & even)
    def _():
        issue(acc0, sem0)

    @pl.when((f == nf - 1) & jnp.logical_not(even))
    def _():
        issue(acc1, sem1)

    @pl.when((f == nf - 1) & (e == pl.num_programs(0) - 1))
    def _():
        pltpu.make_async_copy(acc0, out_ref.at[pl.ds(0, _CAP)], sem0).wait()
        pltpu.make_async_copy(acc1, out_ref.at[pl.ds(0, _CAP)], sem1).wait()


def _sc_gather_rows(table, idx):
    mesh = plsc.VectorSubcoreMesh(core_axis_name="c", subcore_axis_name="s")
    nch = _RW // _CH

    @functools.partial(
        pl.kernel,
        out_type=jax.ShapeDtypeStruct((_S, _M), jnp.float32),
        mesh=mesh,
        scratch_types=[
            pltpu.VMEM((_RW,), jnp.int32),
            pltpu.VMEM((_CH, _M), jnp.float32),
            pltpu.VMEM((_CH, _M), jnp.float32),
            pltpu.SemaphoreType.DMA,
            pltpu.SemaphoreType.DMA,
            pltpu.SemaphoreType.DMA,
            pltpu.SemaphoreType.DMA,
        ],
    )
    def k(table_hbm, idx_hbm, out_hbm, ix, b0, b1, sg0, sg1, sw0, sw1):
        wid = lax.axis_index("s") * _NC + lax.axis_index("c")
        base = wid * _RW
        bufs, sgs, sws = (b0, b1), (sg0, sg1), (sw0, sw1)
        pltpu.sync_copy(idx_hbm.at[pl.ds(base, _RW)], ix)

        def gather(c):
            return pltpu.async_copy(
                table_hbm.at[ix.at[pl.ds(c * _CH, _CH)]], bufs[c % 2], sgs[c % 2])

        gh = {c: gather(c) for c in range(min(2, nch))}
        wh = {}
        for c in range(nch):
            p = c % 2
            gh[c].wait()
            wh[c] = pltpu.async_copy(
                bufs[p], out_hbm.at[pl.ds(base + c * _CH, _CH)], sws[p])
            if c + 2 < nch:
                wh[c].wait()
                gh[c + 2] = gather(c + 2)
        for c in (nch - 2, nch - 1):
            wh[c].wait()

    return k(table, idx)


def _gating_call(x, wg):
    return pl.pallas_call(
        _gating_body,
        out_shape=[
            jax.ShapeDtypeStruct((_S, 1), jnp.int32),
            jax.ShapeDtypeStruct((_S, 1), jnp.float32),
            jax.ShapeDtypeStruct((_S, 1), jnp.int32),
            jax.ShapeDtypeStruct((1, 1), jnp.float32),
        ],
    )(x, wg)


def _ffn_call(scat, gslot, disp, w1, w2):
    nf = _F // _FB
    grid_spec = pltpu.PrefetchScalarGridSpec(
        num_scalar_prefetch=1,
        grid=(_E, nf),
        in_specs=[
            pl.BlockSpec((_CAP, 1), lambda e, f, *_: (e, 0)),
            pl.BlockSpec((_CAP, _M), lambda e, f, *_: (e, 0)),
            pl.BlockSpec((1, _M, _FB), lambda e, f, *_: (e, 0, f)),
            pl.BlockSpec((1, _FB, _M), lambda e, f, *_: (e, f, 0)),
        ],
        out_specs=pl.BlockSpec(memory_space=pl.ANY),
        scratch_shapes=[
            pltpu.VMEM((_CAP, _M), jnp.float32),
            pltpu.VMEM((_CAP, _M), jnp.float32),
            pltpu.SemaphoreType.DMA,
            pltpu.SemaphoreType.DMA,
        ],
    )
    return pl.pallas_call(
        _ffn_body,
        grid_spec=grid_spec,
        out_shape=jax.ShapeDtypeStruct((_S, _M), jnp.float32),
        compiler_params=pltpu.CompilerParams(
            vmem_limit_bytes=63 * 1024 * 1024),
    )(scat, gslot, disp, w1, w2)


def kernel(hidden_states, wg, w1, w2):
    x = hidden_states.reshape(-1, _M)
    src, gslot, scat, laux = _gating_call(x, wg)
    disp = _sc_gather_rows(x, src.reshape(_S))
    out = _ffn_call(scat.reshape(_S), gslot, disp, w1, w2)
    return out, laux.reshape(())
